# baseline (speedup 1.0000x reference)
import functools

import jax
import jax.numpy as jnp
from jax import lax
from jax.experimental import pallas as pl
from jax.experimental.pallas import tpu as pltpu
from jax.experimental.pallas import tpu_sc as plsc

BEAM = 16
VOCAB = 1_000_000
CHUNK = 4096
NC = (VOCAB + CHUNK - 1) // CHUNK
SUBC = 32
NCB = (NC + SUBC - 1) // SUBC
NCP = 256
NSEL = 16
GPC = 8
LANES = 128
POOL_ROWS = NSEL * CHUNK // LANES
BLK_ROWS = 64
NBLK = POOL_ROWS // BLK_ROWS
SEQ = 2048
EOS = 2
IBIG = 0x7FFFFFFF


def _scan_body(d_ref, ids_ref, m_scr):
    c = pl.program_id(0)
    x = d_ref[:, 0, :]
    cio = lax.broadcasted_iota(jnp.int32, (BEAM, NCP), 1)

    @pl.when(c == 0)
    def _():
        m_scr[...] = jnp.full((BEAM, NCP), -1.0, jnp.float32)

    def chunk_maxes(xv):
        out = m_scr[...]
        for i in range(SUBC):
            mx = jnp.max(xv[:, i * CHUNK:(i + 1) * CHUNK], axis=1,
                         keepdims=True)
            out = jnp.where(cio == c * SUBC + i, mx, out)
        return out

    @pl.when(c < NCB - 1)
    def _():
        m_scr[...] = chunk_maxes(x)

    @pl.when(c == NCB - 1)
    def _():
        lio = lax.broadcasted_iota(jnp.int32, (BEAM, SUBC * CHUNK), 1)
        xm = jnp.where(c * SUBC * CHUNK + lio < VOCAB, x, -1.0)
        M = chunk_maxes(xm)
        sel = jnp.zeros((BEAM, NCP), jnp.bool_)
        for _ in range(NSEL):
            row_mx = jnp.max(M, axis=1, keepdims=True)
            cid = jnp.min(jnp.where(M == row_mx, cio, IBIG), axis=1,
                          keepdims=True)
            sel = sel | (cio == cid)
            M = jnp.where(cio == cid, -2.0, M)
        kio = lax.broadcasted_iota(jnp.int32, (BEAM, NSEL), 1)
        ids_acc = jnp.zeros((BEAM, NSEL), jnp.int32)
        for k in range(NSEL):
            cid = jnp.min(jnp.where(sel, cio, IBIG), axis=1, keepdims=True)
            ids_acc = jnp.where(kio == k,
                                jnp.broadcast_to(cid, (BEAM, NSEL)), ids_acc)
            sel = sel & (cio != cid)
        ids_ref[...] = ids_acc


def _gather_body(ids_sref, *refs):
    b = pl.program_id(0)
    h = pl.program_id(1)
    out = refs[GPC]
    lio = lax.broadcasted_iota(jnp.int32, (1, CHUNK), 1)
    for i in range(GPC):
        cid = ids_sref[b, h * GPC + i]
        x = refs[i][:, 0, :]
        x = jnp.where(cid * CHUNK + lio < VOCAB, x, -1.0)
        out[0, pl.ds(i * (CHUNK // LANES), CHUNK // LANES), :] = (
            x.reshape(CHUNK // LANES, LANES))


def _extract_merge_body(ids_sref, d_ref, ids_ref, gen_ref, scores_ref,
                        step_ref, out_scores, out_lens, out_br, out_bidx,
                        p3_scr):
    rmax = jnp.max(d_ref[...], axis=2)
    rio = lax.broadcasted_iota(jnp.int32, (BEAM, POOL_ROWS), 1)
    k_io = lax.broadcasted_iota(jnp.int32, (1, BEAM), 1)
    col_io = lax.broadcasted_iota(jnp.int32, (BEAM, BEAM), 1)

    sel = jnp.zeros((BEAM, POOL_ROWS), jnp.bool_)
    Mr = rmax
    for _ in range(BEAM):
        m = jnp.max(Mr, axis=1, keepdims=True)
        rid = jnp.min(jnp.where(Mr == m, rio, IBIG), axis=1, keepdims=True)
        sel = sel | (rio == rid)
        Mr = jnp.where(rio == rid, -2.0, Mr)
    rid_mat = jnp.zeros((BEAM, BEAM), jnp.int32)
    for k in range(BEAM):
        rid = jnp.min(jnp.where(sel, rio, IBIG), axis=1, keepdims=True)
        rid_mat = jnp.where(col_io == k,
                            jnp.broadcast_to(rid, (BEAM, BEAM)), rid_mat)
        sel = sel & (rio != rid)

    rid_masked = [jnp.where(col_io == k, rid_mat, IBIG) for k in range(BEAM)]
    for b in range(BEAM):
        for k in range(BEAM):
            rs = jnp.min(rid_masked[k][b, :])
            p3_scr[pl.ds(b, 1), pl.ds(k, 1), :] = (
                d_ref[b, pl.ds(rs, 1), :].reshape(1, 1, LANES))

    P = p3_scr[...]
    lio3 = (lax.broadcasted_iota(jnp.int32, (BEAM, BEAM, LANES), 1) * LANES
            + lax.broadcasted_iota(jnp.int32, (BEAM, BEAM, LANES), 2))
    ids_mat = ids_ref[...]

    vals = jnp.zeros((BEAM, BEAM), jnp.float32)
    idxs = jnp.zeros((BEAM, BEAM), jnp.int32)
    for k in range(BEAM):
        m = jnp.max(P, axis=(1, 2), keepdims=True)
        p = jnp.min(jnp.where(P == m, lio3, IBIG), axis=(1, 2),
                    keepdims=True)
        p2 = p[:, :, 0]
        k3 = p2 // LANES
        l3 = lax.rem(p2, LANES)
        rowid = jnp.min(jnp.where(col_io == k3, rid_mat, IBIG), axis=1,
                        keepdims=True)
        j = rowid // (CHUNK // LANES)
        cid = jnp.min(jnp.where(col_io == j, ids_mat, IBIG), axis=1,
                      keepdims=True)
        vocab = (cid * (CHUNK // LANES)
                 + lax.rem(rowid, CHUNK // LANES)) * LANES + l3
        vals = jnp.where(col_io == k, jnp.broadcast_to(m[:, :, 0], vals.shape),
                         vals)
        idxs = jnp.where(col_io == k, jnp.broadcast_to(vocab, idxs.shape),
                         idxs)
        P = jnp.where(lio3 == p, -2.0, P)

    s = jnp.log(vals) + scores_ref[...]
    f_io = (lax.broadcasted_iota(jnp.int32, (BEAM, BEAM), 0) * BEAM
            + lax.broadcasted_iota(jnp.int32, (BEAM, BEAM), 1))

    scores_acc = jnp.zeros((1, BEAM), jnp.float32)
    lens_acc = jnp.zeros((1, BEAM), jnp.int32)
    picks = []
    for k in range(BEAM):
        m = jnp.max(s)
        fidx = jnp.min(jnp.where(s == m, f_io, IBIG))
        bidx = jnp.min(jnp.where(f_io == fidx, idxs, IBIG))
        picks.append((fidx // BEAM, bidx))
        scores_acc = jnp.where(k_io == k, m, scores_acc)
        s = jnp.where(f_io == fidx, -jnp.inf, s)
    out_scores[...] = scores_acc

    st = step_ref[0, 0]
    pos = lax.broadcasted_iota(jnp.int32, (BEAM, SEQ), 1)
    gen = gen_ref[...]
    eo = gen == EOS
    pref = jnp.min(jnp.where(eo & (pos < st), pos + 1, SEQ), axis=1,
                   keepdims=True)
    suf = jnp.min(jnp.where(eo & (pos > st), pos + 1, SEQ), axis=1,
                  keepdims=True)
    rio16 = lax.broadcasted_iota(jnp.int32, (BEAM, 1), 0)
    br_acc = jnp.zeros((1, BEAM), jnp.int32)
    bidx_acc = jnp.zeros((1, BEAM), jnp.int32)
    for k in range(BEAM):
        r, bidx = picks[k]
        p_r = jnp.min(jnp.where(rio16 == r, pref, SEQ))
        s_k = jnp.min(jnp.where(rio16 == k, suf, SEQ))
        e_at = jnp.where(bidx == EOS, st + 1, SEQ)
        sl = jnp.minimum(jnp.minimum(p_r, s_k), e_at)
        lens_acc = jnp.where(k_io == k, sl, lens_acc)
        br_acc = jnp.where(k_io == k, r, br_acc)
        bidx_acc = jnp.where(k_io == k, bidx, bidx_acc)
    out_lens[...] = lens_acc
    out_br[...] = br_acc
    out_bidx[...] = bidx_acc


NW = 32
COLS_W = SEQ // NW
SC_L = 16


@functools.partial(
    pl.kernel,
    mesh=plsc.VectorSubcoreMesh(core_axis_name="c", subcore_axis_name="s"),
    out_type=jax.ShapeDtypeStruct((BEAM, SEQ), jnp.int32),
    scratch_types=[pltpu.VMEM((BEAM,), jnp.int32),
                   pltpu.VMEM((BEAM, SEQ), jnp.int32),
                   pltpu.VMEM((BEAM, SEQ), jnp.int32),
                   pltpu.VMEM((BEAM,), jnp.int32),
                   pltpu.VMEM((BEAM, BEAM), jnp.int32),
                   pltpu.SemaphoreType.DMA],
)
def _sc_reorder(gen_hbm, br_hbm, bb_hbm, sv_hbm, out_hbm,
                idx_v, rows_v, orig_v, sv_v, bb_v, sem):
    wid = lax.axis_index("s") * 2 + lax.axis_index("c")
    pltpu.sync_copy(br_hbm, idx_v)
    pltpu.async_copy(gen_hbm.at[idx_v], rows_v, sem).wait()
    pltpu.sync_copy(gen_hbm, orig_v)
    pltpu.sync_copy(sv_hbm, sv_v)
    pltpu.sync_copy(bb_hbm, bb_v)
    st = sv_v[...]
    base = wid * COLS_W
    lane = lax.iota(jnp.int32, SC_L)
    for r in range(BEAM):
        bb = bb_v[r, pl.ds(0, SC_L)]
        for j in range(COLS_W // SC_L):
            c0 = base + j * SC_L
            posv = lane + c0
            g = rows_v[r, pl.ds(c0, SC_L)]
            o = orig_v[r, pl.ds(c0, SC_L)]
            m = jnp.where(posv < st, g, o)
            m = jnp.where(posv == st, bb, m)
            orig_v[r, pl.ds(c0, SC_L)] = m
    for r in range(BEAM):
        pltpu.sync_copy(orig_v.at[pl.ds(r, 1), pl.ds(base, COLS_W)],
                        out_hbm.at[pl.ds(r, 1), pl.ds(base, COLS_W)])


def kernel(dec_output, scores, gen_seq, step):
    ids = pl.pallas_call(
        _scan_body,
        grid=(NCB,),
        in_specs=[pl.BlockSpec((BEAM, 1, SUBC * CHUNK), lambda c: (0, 0, c))],
        out_specs=pl.BlockSpec((BEAM, NSEL), lambda c: (0, 0)),
        out_shape=jax.ShapeDtypeStruct((BEAM, NSEL), jnp.int32),
        scratch_shapes=[pltpu.VMEM((BEAM, NCP), jnp.float32)],
    )(dec_output)

    grid_spec = pltpu.PrefetchScalarGridSpec(
        num_scalar_prefetch=1,
        grid=(BEAM, NSEL // GPC),
        in_specs=[pl.BlockSpec((1, 1, CHUNK),
                               (lambda b, h, ids_m, i=i:
                                (b, 0, ids_m[b, h * GPC + i])))
                  for i in range(GPC)],
        out_specs=pl.BlockSpec((1, GPC * CHUNK // LANES, LANES),
                               lambda b, h, ids_m: (b, h, 0)),
    )
    pool = pl.pallas_call(
        _gather_body,
        grid_spec=grid_spec,
        out_shape=jax.ShapeDtypeStruct((BEAM, POOL_ROWS, LANES), jnp.float32),
    )(ids, *([dec_output] * GPC))

    step_arr = jnp.asarray(step, jnp.int32).reshape(1, 1)
    scores2 = scores.reshape(BEAM, 1)
    grid_spec_d = pltpu.PrefetchScalarGridSpec(
        num_scalar_prefetch=1,
        grid=(1,),
        in_specs=[pl.BlockSpec((BEAM, POOL_ROWS, LANES),
                               lambda c, ids_m: (0, 0, 0)),
                  pl.BlockSpec((BEAM, BEAM), lambda c, ids_m: (0, 0)),
                  pl.BlockSpec((BEAM, SEQ), lambda c, ids_m: (0, 0)),
                  pl.BlockSpec((BEAM, 1), lambda c, ids_m: (0, 0)),
                  pl.BlockSpec(memory_space=pltpu.SMEM)],
        out_specs=[pl.BlockSpec((1, BEAM), lambda c, ids_m: (0, 0)),
                   pl.BlockSpec((1, BEAM), lambda c, ids_m: (0, 0)),
                   pl.BlockSpec((1, BEAM), lambda c, ids_m: (0, 0)),
                   pl.BlockSpec((1, BEAM), lambda c, ids_m: (0, 0))],
        scratch_shapes=[pltpu.VMEM((BEAM, BEAM, LANES), jnp.float32)],
    )
    scores_new, seq_lens, br, bidx = pl.pallas_call(
        _extract_merge_body,
        grid_spec=grid_spec_d,
        out_shape=[jax.ShapeDtypeStruct((1, BEAM), jnp.float32),
                   jax.ShapeDtypeStruct((1, BEAM), jnp.int32),
                   jax.ShapeDtypeStruct((1, BEAM), jnp.int32),
                   jax.ShapeDtypeStruct((1, BEAM), jnp.int32)],
    )(ids, pool, ids, gen_seq, scores2, step_arr)

    bb = jnp.broadcast_to(bidx.reshape(BEAM, 1), (BEAM, BEAM))
    sv = jnp.full((BEAM,), jnp.asarray(step, jnp.int32))
    new_gen = _sc_reorder(gen_seq, br.reshape(BEAM), bb, sv)
    return new_gen, scores_new.reshape(BEAM), seq_lens.reshape(BEAM)


# speedup vs baseline: 1.0302x; 1.0302x over previous
import functools

import jax
import jax.numpy as jnp
from jax import lax
from jax.experimental import pallas as pl
from jax.experimental.pallas import tpu as pltpu
from jax.experimental.pallas import tpu_sc as plsc

BEAM = 16
VOCAB = 1_000_000
CHUNK = 4096
NC = (VOCAB + CHUNK - 1) // CHUNK
SUBC = 32
NCB = (NC + SUBC - 1) // SUBC
NCP = 256
NSEL = 16
GPC = 8
LANES = 128
POOL_ROWS = NSEL * CHUNK // LANES
BLK_ROWS = 64
NBLK = POOL_ROWS // BLK_ROWS
SEQ = 2048
EOS = 2
IBIG = 0x7FFFFFFF


def _scan_body(d_ref, ids_ref, m_scr):
    c = pl.program_id(0)
    x = d_ref[:, 0, :]
    cio = lax.broadcasted_iota(jnp.int32, (BEAM, NCP), 1)

    @pl.when(c == 0)
    def _():
        m_scr[...] = jnp.full((BEAM, NCP), -1.0, jnp.float32)

    def chunk_maxes(xv):
        out = m_scr[...]
        for i in range(SUBC):
            mx = jnp.max(xv[:, i * CHUNK:(i + 1) * CHUNK], axis=1,
                         keepdims=True)
            out = jnp.where(cio == c * SUBC + i, mx, out)
        return out

    @pl.when(c < NCB - 1)
    def _():
        m_scr[...] = chunk_maxes(x)

    @pl.when(c == NCB - 1)
    def _():
        lio = lax.broadcasted_iota(jnp.int32, (BEAM, SUBC * CHUNK), 1)
        xm = jnp.where(c * SUBC * CHUNK + lio < VOCAB, x, -1.0)
        M = chunk_maxes(xm)
        sel = jnp.zeros((BEAM, NCP), jnp.bool_)
        for _ in range(NSEL):
            row_mx = jnp.max(M, axis=1, keepdims=True)
            cid = jnp.min(jnp.where(M == row_mx, cio, IBIG), axis=1,
                          keepdims=True)
            sel = sel | (cio == cid)
            M = jnp.where(cio == cid, -2.0, M)
        kio = lax.broadcasted_iota(jnp.int32, (BEAM, NSEL), 1)
        ids_acc = jnp.zeros((BEAM, NSEL), jnp.int32)
        for k in range(NSEL):
            cid = jnp.min(jnp.where(sel, cio, IBIG), axis=1, keepdims=True)
            ids_acc = jnp.where(kio == k,
                                jnp.broadcast_to(cid, (BEAM, NSEL)), ids_acc)
            sel = sel & (cio != cid)
        ids_ref[...] = ids_acc


def _gather_body(ids_sref, *refs):
    b = pl.program_id(0)
    h = pl.program_id(1)
    out = refs[GPC]
    lio = lax.broadcasted_iota(jnp.int32, (1, CHUNK), 1)
    for i in range(GPC):
        cid = ids_sref[b, h * GPC + i]
        x = refs[i][:, 0, :]
        x = jnp.where(cid * CHUNK + lio < VOCAB, x, -1.0)
        out[0, pl.ds(i * (CHUNK // LANES), CHUNK // LANES), :] = (
            x.reshape(CHUNK // LANES, LANES))


def _extract_merge_body(ids_sref, d_ref, ids_ref, gen_ref, scores_ref,
                        step_ref, out_scores, out_lens, out_br, out_bidx,
                        p3_scr):
    rmax = jnp.max(d_ref[...], axis=2)
    rio = lax.broadcasted_iota(jnp.int32, (BEAM, POOL_ROWS), 1)
    k_io = lax.broadcasted_iota(jnp.int32, (1, BEAM), 1)
    col_io = lax.broadcasted_iota(jnp.int32, (BEAM, BEAM), 1)

    sel = jnp.zeros((BEAM, POOL_ROWS), jnp.bool_)
    Mr = rmax
    for _ in range(BEAM):
        m = jnp.max(Mr, axis=1, keepdims=True)
        rid = jnp.min(jnp.where(Mr == m, rio, IBIG), axis=1, keepdims=True)
        sel = sel | (rio == rid)
        Mr = jnp.where(rio == rid, -2.0, Mr)
    rid_mat = jnp.zeros((BEAM, BEAM), jnp.int32)
    for k in range(BEAM):
        rid = jnp.min(jnp.where(sel, rio, IBIG), axis=1, keepdims=True)
        rid_mat = jnp.where(col_io == k,
                            jnp.broadcast_to(rid, (BEAM, BEAM)), rid_mat)
        sel = sel & (rio != rid)

    rid_masked = [jnp.where(col_io == k, rid_mat, IBIG) for k in range(BEAM)]
    for b in range(BEAM):
        for k in range(BEAM):
            rs = jnp.min(rid_masked[k][b, :])
            p3_scr[pl.ds(b, 1), pl.ds(k, 1), :] = (
                d_ref[b, pl.ds(rs, 1), :].reshape(1, 1, LANES))

    P = p3_scr[...]
    lio3 = (lax.broadcasted_iota(jnp.int32, (BEAM, BEAM, LANES), 1) * LANES
            + lax.broadcasted_iota(jnp.int32, (BEAM, BEAM, LANES), 2))
    ids_mat = ids_ref[...]

    vals = jnp.zeros((BEAM, BEAM), jnp.float32)
    idxs = jnp.zeros((BEAM, BEAM), jnp.int32)
    for k in range(BEAM):
        m = jnp.max(P, axis=(1, 2), keepdims=True)
        p = jnp.min(jnp.where(P == m, lio3, IBIG), axis=(1, 2),
                    keepdims=True)
        p2 = p[:, :, 0]
        k3 = p2 // LANES
        l3 = lax.rem(p2, LANES)
        rowid = jnp.min(jnp.where(col_io == k3, rid_mat, IBIG), axis=1,
                        keepdims=True)
        j = rowid // (CHUNK // LANES)
        cid = jnp.min(jnp.where(col_io == j, ids_mat, IBIG), axis=1,
                      keepdims=True)
        vocab = (cid * (CHUNK // LANES)
                 + lax.rem(rowid, CHUNK // LANES)) * LANES + l3
        vals = jnp.where(col_io == k, jnp.broadcast_to(m[:, :, 0], vals.shape),
                         vals)
        idxs = jnp.where(col_io == k, jnp.broadcast_to(vocab, idxs.shape),
                         idxs)
        P = jnp.where(lio3 == p, -2.0, P)

    s = jnp.log(vals) + scores_ref[...]
    f_io = (lax.broadcasted_iota(jnp.int32, (BEAM, BEAM), 0) * BEAM
            + lax.broadcasted_iota(jnp.int32, (BEAM, BEAM), 1))

    scores_acc = jnp.zeros((1, BEAM), jnp.float32)
    lens_acc = jnp.zeros((1, BEAM), jnp.int32)
    picks = []
    for k in range(BEAM):
        m = jnp.max(s)
        fidx = jnp.min(jnp.where(s == m, f_io, IBIG))
        bidx = jnp.min(jnp.where(f_io == fidx, idxs, IBIG))
        picks.append((fidx // BEAM, bidx))
        scores_acc = jnp.where(k_io == k, m, scores_acc)
        s = jnp.where(f_io == fidx, -jnp.inf, s)
    out_scores[...] = scores_acc

    st = step_ref[0, 0]
    pos = lax.broadcasted_iota(jnp.int32, (BEAM, SEQ), 1)
    gen = gen_ref[...]
    eo = gen == EOS
    pref = jnp.min(jnp.where(eo & (pos < st), pos + 1, SEQ), axis=1,
                   keepdims=True)
    suf = jnp.min(jnp.where(eo & (pos > st), pos + 1, SEQ), axis=1,
                  keepdims=True)
    rio16 = lax.broadcasted_iota(jnp.int32, (BEAM, 1), 0)
    br_acc = jnp.zeros((1, BEAM), jnp.int32)
    bidx_acc = jnp.zeros((1, BEAM), jnp.int32)
    for k in range(BEAM):
        r, bidx = picks[k]
        p_r = jnp.min(jnp.where(rio16 == r, pref, SEQ))
        s_k = jnp.min(jnp.where(rio16 == k, suf, SEQ))
        e_at = jnp.where(bidx == EOS, st + 1, SEQ)
        sl = jnp.minimum(jnp.minimum(p_r, s_k), e_at)
        lens_acc = jnp.where(k_io == k, sl, lens_acc)
        br_acc = jnp.where(k_io == k, r, br_acc)
        bidx_acc = jnp.where(k_io == k, bidx, bidx_acc)
    out_lens[...] = lens_acc
    out_br[...] = br_acc
    out_bidx[...] = bidx_acc


SC_L = 16
HALF = SEQ // 2


@functools.partial(
    pl.kernel,
    mesh=plsc.VectorSubcoreMesh(core_axis_name="c", subcore_axis_name="s"),
    out_type=jax.ShapeDtypeStruct((BEAM, SEQ), jnp.int32),
    scratch_types=[pltpu.VMEM((BEAM,), jnp.int32),
                   pltpu.VMEM((BEAM, SEQ), jnp.int32),
                   pltpu.VMEM((1, HALF), jnp.int32),
                   pltpu.VMEM((BEAM,), jnp.int32),
                   pltpu.VMEM((BEAM, BEAM), jnp.int32),
                   pltpu.SemaphoreType.DMA],
)
def _sc_reorder(gen_hbm, br_hbm, bb_hbm, sv_hbm, out_hbm,
                idx_v, rows_v, orig_v, sv_v, bb_v, sem):
    wid = lax.axis_index("s") * 2 + lax.axis_index("c")
    r = wid // 2
    h = lax.rem(wid, 2)
    pltpu.sync_copy(br_hbm, idx_v)
    pltpu.async_copy(gen_hbm.at[idx_v], rows_v, sem).wait()
    pltpu.sync_copy(gen_hbm.at[pl.ds(r, 1), pl.ds(h * HALF, HALF)], orig_v)
    pltpu.sync_copy(sv_hbm, sv_v)
    pltpu.sync_copy(bb_hbm, bb_v)
    st = sv_v[...]
    lane = lax.iota(jnp.int32, SC_L)
    bb = bb_v[r, pl.ds(0, SC_L)]
    for j in range(HALF // SC_L):
        c0 = h * HALF + j * SC_L
        posv = lane + c0
        g = rows_v[r, pl.ds(c0, SC_L)]
        o = orig_v[0, pl.ds(j * SC_L, SC_L)]
        m = jnp.where(posv < st, g, o)
        m = jnp.where(posv == st, bb, m)
        orig_v[0, pl.ds(j * SC_L, SC_L)] = m
    pltpu.sync_copy(orig_v,
                    out_hbm.at[pl.ds(r, 1), pl.ds(h * HALF, HALF)])


def kernel(dec_output, scores, gen_seq, step):
    ids = pl.pallas_call(
        _scan_body,
        grid=(NCB,),
        in_specs=[pl.BlockSpec((BEAM, 1, SUBC * CHUNK), lambda c: (0, 0, c))],
        out_specs=pl.BlockSpec((BEAM, NSEL), lambda c: (0, 0)),
        out_shape=jax.ShapeDtypeStruct((BEAM, NSEL), jnp.int32),
        scratch_shapes=[pltpu.VMEM((BEAM, NCP), jnp.float32)],
    )(dec_output)

    grid_spec = pltpu.PrefetchScalarGridSpec(
        num_scalar_prefetch=1,
        grid=(BEAM, NSEL // GPC),
        in_specs=[pl.BlockSpec((1, 1, CHUNK),
                               (lambda b, h, ids_m, i=i:
                                (b, 0, ids_m[b, h * GPC + i])))
                  for i in range(GPC)],
        out_specs=pl.BlockSpec((1, GPC * CHUNK // LANES, LANES),
                               lambda b, h, ids_m: (b, h, 0)),
    )
    pool = pl.pallas_call(
        _gather_body,
        grid_spec=grid_spec,
        out_shape=jax.ShapeDtypeStruct((BEAM, POOL_ROWS, LANES), jnp.float32),
    )(ids, *([dec_output] * GPC))

    step_arr = jnp.asarray(step, jnp.int32).reshape(1, 1)
    scores2 = scores.reshape(BEAM, 1)
    grid_spec_d = pltpu.PrefetchScalarGridSpec(
        num_scalar_prefetch=1,
        grid=(1,),
        in_specs=[pl.BlockSpec((BEAM, POOL_ROWS, LANES),
                               lambda c, ids_m: (0, 0, 0)),
                  pl.BlockSpec((BEAM, BEAM), lambda c, ids_m: (0, 0)),
                  pl.BlockSpec((BEAM, SEQ), lambda c, ids_m: (0, 0)),
                  pl.BlockSpec((BEAM, 1), lambda c, ids_m: (0, 0)),
                  pl.BlockSpec(memory_space=pltpu.SMEM)],
        out_specs=[pl.BlockSpec((1, BEAM), lambda c, ids_m: (0, 0)),
                   pl.BlockSpec((1, BEAM), lambda c, ids_m: (0, 0)),
                   pl.BlockSpec((1, BEAM), lambda c, ids_m: (0, 0)),
                   pl.BlockSpec((1, BEAM), lambda c, ids_m: (0, 0))],
        scratch_shapes=[pltpu.VMEM((BEAM, BEAM, LANES), jnp.float32)],
    )
    scores_new, seq_lens, br, bidx = pl.pallas_call(
        _extract_merge_body,
        grid_spec=grid_spec_d,
        out_shape=[jax.ShapeDtypeStruct((1, BEAM), jnp.float32),
                   jax.ShapeDtypeStruct((1, BEAM), jnp.int32),
                   jax.ShapeDtypeStruct((1, BEAM), jnp.int32),
                   jax.ShapeDtypeStruct((1, BEAM), jnp.int32)],
    )(ids, pool, ids, gen_seq, scores2, step_arr)

    bb = jnp.broadcast_to(bidx.reshape(BEAM, 1), (BEAM, BEAM))
    sv = jnp.full((BEAM,), jnp.asarray(step, jnp.int32))
    new_gen = _sc_reorder(gen_seq, br.reshape(BEAM), bb, sv)
    return new_gen, scores_new.reshape(BEAM), seq_lens.reshape(BEAM)
